# baseline (speedup 1.0000x reference)
import numpy as np
import jax
import jax.numpy as jnp
from jax.experimental import pallas as pl
from jax.experimental.pallas import tpu as pltpu
from jax.experimental.pallas import tpu_sc as plsc

BS, LENGTH, D_Y, D_X, D_MODEL = 8, 512, 8, 7, 512
T2V_IN = D_X + 1
T2V_K = D_MODEL // T2V_IN
N_TOK = D_Y * LENGTH
REP = 128

_E7 = np.repeat(np.eye(T2V_IN, dtype=np.float32), T2V_K, axis=1)[:D_X]
_E8 = np.repeat(np.eye(T2V_IN, dtype=np.float32), T2V_K, axis=1)[D_X:]
_LIN_MASK = ((np.arange(D_MODEL, dtype=np.int32) % T2V_K) == 0).astype(
    np.int32)[None, :]

_INV_PI = float(1.0 / np.pi)
_PI = float(np.pi)


def _fast_sin(a):
    q = jnp.round(a * _INV_PI)
    r = a - q * _PI
    r2 = r * r
    p = r * (1.0 + r2 * (-0.166666667 + r2 * (0.0083333310
                                              + r2 * (-0.000198408))))
    sign = jax.lax.shift_left(q.astype(jnp.int32) & 1, 31)
    return jax.lax.bitcast_convert_type(
        jax.lax.bitcast_convert_type(p, jnp.int32) ^ sign, jnp.float32)


def _row(ref):
    return jnp.concatenate([ref[i:i + 1, :] for i in range(T2V_IN)], axis=1)


def _tc_body(x_ref, y_ref, e7_ref, e8_ref, m_ref, w_ref, b_ref, yb_ref,
             g_ref, bW_ref, out_ref, buf_ref, sems):
    i = pl.program_id(0)
    xtb = x_ref[:, i, :]
    lp = (jax.lax.broadcasted_iota(jnp.int32, (LENGTH, 1), 0)
          .astype(jnp.float32) * (1.0 / LENGTH))
    xce = jax.lax.dot_general(
        xtb, e7_ref[...], (((0,), (0,)), ((), ())),
        precision=jax.lax.Precision.HIGHEST) + lp * e8_ref[...]
    a = xce * _row(w_ref) + _row(b_ref)
    s = jnp.where(m_ref[...] != 0, a, _fast_sin(a))
    w1 = bW_ref[1:1 + D_MODEL, :]
    c = jnp.reshape(yb_ref[...], (1, D_MODEL)) + g_ref[1:2, :]
    t = jax.lax.dot(s, w1) + c
    w0 = bW_ref[0:1, :]
    yb = jnp.transpose(y_ref[0])
    copies = []
    for v in range(D_Y):
        prev = pltpu.make_async_copy(
            buf_ref.at[v],
            out_ref.at[i - 1, pl.ds(v * LENGTH, LENGTH), :], sems.at[v])

        @pl.when(i > 0)
        def _():
            prev.wait()

        yv = yb[:, v:v + 1]
        buf_ref[v] = t + yv * w0
        cp = pltpu.make_async_copy(
            buf_ref.at[v],
            out_ref.at[i, pl.ds(v * LENGTH, LENGTH), :], sems.at[v])
        cp.start()
        copies.append(cp)

    @pl.when(i == BS - 1)
    def _():
        for cp in copies:
            cp.wait()


def _sc_body(tab_ref, vemb_ref, vidx_ref, rep_ref, idx_ref, sem):
    core = jax.lax.axis_index("c")
    sub = jax.lax.axis_index("s")
    tec = core * 16 + sub
    v = jax.lax.rem(tec, D_Y)
    b0 = jax.lax.div(tec, D_Y)
    pltpu.async_copy(tab_ref.at[v], rep_ref.at[0], sem).wait()
    vvec = jnp.broadcast_to(v, (16,)).astype(jnp.int32)

    @pl.loop(0, D_MODEL, step=16)
    def _(i):
        val = rep_ref[0, pl.ds(i, 16)]
        for r in range(1, REP):
            rep_ref[r, pl.ds(i, 16)] = val

    @pl.loop(0, LENGTH, step=16)
    def _(i):
        idx_ref[pl.ds(i, 16)] = vvec

    copies = []
    for j in range(2):
        b = b0 + j * 4
        for k in range(LENGTH // REP):
            copies.append(pltpu.async_copy(
                rep_ref,
                vemb_ref.at[b, pl.ds(v * LENGTH + k * REP, REP), :], sem))
        copies.append(pltpu.async_copy(
            idx_ref, vidx_ref.at[b, pl.ds(v * LENGTH, LENGTH)], sem))
    for cp in copies:
        cp.wait()


def _sc_var_outputs(var_emb_table):
    mesh = plsc.VectorSubcoreMesh(core_axis_name="c", subcore_axis_name="s")
    fn = pl.kernel(
        _sc_body,
        out_type=[
            jax.ShapeDtypeStruct((BS, N_TOK, D_MODEL), jnp.float32),
            jax.ShapeDtypeStruct((BS, N_TOK), jnp.int32),
        ],
        mesh=mesh,
        scratch_types=[
            pltpu.VMEM((REP, D_MODEL), jnp.float32),
            pltpu.VMEM((LENGTH,), jnp.int32),
            pltpu.SemaphoreType.DMA,
        ],
    )
    return fn(var_emb_table)


def kernel(y, x, t2v_weight, t2v_bias, y_emb_W, y_emb_b, var_emb_table,
           given_emb_table):
    var_emb, var_idx = _sc_var_outputs(var_emb_table)
    val_time = pl.pallas_call(
        _tc_body,
        grid=(BS,),
        in_specs=[
            pl.BlockSpec((D_X, BS, LENGTH), lambda b: (0, 0, 0)),
            pl.BlockSpec((1, D_Y, LENGTH), lambda b: (b, 0, 0)),
            pl.BlockSpec((D_X, D_MODEL), lambda b: (0, 0)),
            pl.BlockSpec((1, D_MODEL), lambda b: (0, 0)),
            pl.BlockSpec((1, D_MODEL), lambda b: (0, 0)),
            pl.BlockSpec((T2V_IN, T2V_K), lambda b: (0, 0)),
            pl.BlockSpec((T2V_IN, T2V_K), lambda b: (0, 0)),
            pl.BlockSpec((D_MODEL,), lambda b: (0,)),
            pl.BlockSpec((2, D_MODEL), lambda b: (0, 0)),
            pl.BlockSpec((1 + D_MODEL, D_MODEL), lambda b: (0, 0)),
        ],
        out_specs=pl.BlockSpec(memory_space=pl.ANY),
        out_shape=jax.ShapeDtypeStruct((BS, N_TOK, D_MODEL), jnp.float32),
        scratch_shapes=[
            pltpu.VMEM((D_Y, LENGTH, D_MODEL), jnp.float32),
            pltpu.SemaphoreType.DMA((D_Y,)),
        ],
    )(jnp.transpose(x, (2, 0, 1)), jnp.transpose(y, (0, 2, 1)),
      _E7, _E8, _LIN_MASK, t2v_weight, t2v_bias, y_emb_b, given_emb_table,
      y_emb_W)
    return val_time, var_emb, var_idx


# speedup vs baseline: 1.0091x; 1.0091x over previous
import numpy as np
import jax
import jax.numpy as jnp
from jax.experimental import pallas as pl
from jax.experimental.pallas import tpu as pltpu
from jax.experimental.pallas import tpu_sc as plsc

BS, LENGTH, D_Y, D_X, D_MODEL = 8, 512, 8, 7, 512
T2V_IN = D_X + 1
T2V_K = D_MODEL // T2V_IN
N_TOK = D_Y * LENGTH
REP = 128

_E7 = np.repeat(np.eye(T2V_IN, dtype=np.float32), T2V_K, axis=1)[:D_X]
_E8 = np.repeat(np.eye(T2V_IN, dtype=np.float32), T2V_K, axis=1)[D_X:]
_LIN_MASK = ((np.arange(D_MODEL, dtype=np.int32) % T2V_K) == 0).astype(
    np.int32)[None, :]

_INV_PI = float(1.0 / np.pi)
_PI = float(np.pi)


def _fast_sin(a):
    q = jnp.round(a * _INV_PI)
    r = a - q * _PI
    r2 = r * r
    p = r * (1.0 + r2 * (-0.166666667 + r2 * (0.0083333310
                                              + r2 * (-0.000198408))))
    sign = jax.lax.shift_left(q.astype(jnp.int32) & 1, 31)
    return jax.lax.bitcast_convert_type(
        jax.lax.bitcast_convert_type(p, jnp.int32) ^ sign, jnp.float32)


def _row(ref):
    return jnp.concatenate([ref[i:i + 1, :] for i in range(T2V_IN)], axis=1)


def _tc_body(x_ref, y_ref, e7_ref, e8_ref, m_ref, w_ref, b_ref, yb_ref,
             g_ref, bW_ref, out_ref):
    xtb = x_ref[:, pl.program_id(0), :]
    lp = (jax.lax.broadcasted_iota(jnp.int32, (LENGTH, 1), 0)
          .astype(jnp.float32) * (1.0 / LENGTH))
    xce = jax.lax.dot_general(
        xtb, e7_ref[...], (((0,), (0,)), ((), ())),
        precision=jax.lax.Precision.HIGHEST) + lp * e8_ref[...]
    a = xce * _row(w_ref) + _row(b_ref)
    s = jnp.where(m_ref[...] != 0, a, _fast_sin(a))
    w1 = bW_ref[1:1 + D_MODEL, :]
    c = jnp.reshape(yb_ref[...], (1, D_MODEL)) + g_ref[1:2, :]
    t = jax.lax.dot(s, w1) + c
    w0 = bW_ref[0:1, :]
    yb = jnp.transpose(y_ref[0])
    for v in range(D_Y):
        yv = yb[:, v:v + 1]
        out_ref[0, v * LENGTH:(v + 1) * LENGTH, :] = t + yv * w0


def _sc_body(tab_ref, vemb_ref, vidx_ref, rep_ref, idx_ref, sem):
    core = jax.lax.axis_index("c")
    sub = jax.lax.axis_index("s")
    tec = core * 16 + sub
    v = jax.lax.rem(tec, D_Y)
    b0 = jax.lax.div(tec, D_Y)
    pltpu.async_copy(tab_ref.at[v], rep_ref.at[0], sem).wait()
    vvec = jnp.broadcast_to(v, (16,)).astype(jnp.int32)

    @pl.loop(0, D_MODEL, step=16)
    def _(i):
        val = rep_ref[0, pl.ds(i, 16)]
        for r in range(1, REP):
            rep_ref[r, pl.ds(i, 16)] = val

    @pl.loop(0, LENGTH, step=16)
    def _(i):
        idx_ref[pl.ds(i, 16)] = vvec

    copies = []
    for j in range(2):
        b = b0 + j * 4
        for k in range(LENGTH // REP):
            copies.append(pltpu.async_copy(
                rep_ref,
                vemb_ref.at[b, pl.ds(v * LENGTH + k * REP, REP), :], sem))
        copies.append(pltpu.async_copy(
            idx_ref, vidx_ref.at[b, pl.ds(v * LENGTH, LENGTH)], sem))
    for cp in copies:
        cp.wait()


def _sc_var_outputs(var_emb_table):
    mesh = plsc.VectorSubcoreMesh(core_axis_name="c", subcore_axis_name="s")
    fn = pl.kernel(
        _sc_body,
        out_type=[
            jax.ShapeDtypeStruct((BS, N_TOK, D_MODEL), jnp.float32),
            jax.ShapeDtypeStruct((BS, N_TOK), jnp.int32),
        ],
        mesh=mesh,
        scratch_types=[
            pltpu.VMEM((REP, D_MODEL), jnp.float32),
            pltpu.VMEM((LENGTH,), jnp.int32),
            pltpu.SemaphoreType.DMA,
        ],
    )
    return fn(var_emb_table)


def kernel(y, x, t2v_weight, t2v_bias, y_emb_W, y_emb_b, var_emb_table,
           given_emb_table):
    var_emb, var_idx = _sc_var_outputs(var_emb_table)
    val_time = pl.pallas_call(
        _tc_body,
        grid=(BS,),
        in_specs=[
            pl.BlockSpec((D_X, BS, LENGTH), lambda b: (0, 0, 0)),
            pl.BlockSpec((1, D_Y, LENGTH), lambda b: (b, 0, 0)),
            pl.BlockSpec((D_X, D_MODEL), lambda b: (0, 0)),
            pl.BlockSpec((1, D_MODEL), lambda b: (0, 0)),
            pl.BlockSpec((1, D_MODEL), lambda b: (0, 0)),
            pl.BlockSpec((T2V_IN, T2V_K), lambda b: (0, 0)),
            pl.BlockSpec((T2V_IN, T2V_K), lambda b: (0, 0)),
            pl.BlockSpec((D_MODEL,), lambda b: (0,)),
            pl.BlockSpec((2, D_MODEL), lambda b: (0, 0)),
            pl.BlockSpec((1 + D_MODEL, D_MODEL), lambda b: (0, 0)),
        ],
        out_specs=pl.BlockSpec((1, N_TOK, D_MODEL), lambda b: (b, 0, 0)),
        out_shape=jax.ShapeDtypeStruct((BS, N_TOK, D_MODEL), jnp.float32),
    )(jnp.transpose(x, (2, 0, 1)), jnp.transpose(y, (0, 2, 1)),
      _E7, _E8, _LIN_MASK, t2v_weight, t2v_bias, y_emb_b, given_emb_table,
      y_emb_W)
    return val_time, var_emb, var_idx


# speedup vs baseline: 1.0110x; 1.0019x over previous
import numpy as np
import jax
import jax.numpy as jnp
from jax.experimental import pallas as pl
from jax.experimental.pallas import tpu as pltpu
from jax.experimental.pallas import tpu_sc as plsc

BS, LENGTH, D_Y, D_X, D_MODEL = 8, 512, 8, 7, 512
T2V_IN = D_X + 1
T2V_K = D_MODEL // T2V_IN
N_TOK = D_Y * LENGTH
REP = 128

_E7 = np.repeat(np.eye(T2V_IN, dtype=np.float32), T2V_K, axis=1)[:D_X]
_E8 = np.repeat(np.eye(T2V_IN, dtype=np.float32), T2V_K, axis=1)[D_X:]
_LIN_MASK = ((np.arange(D_MODEL, dtype=np.int32) % T2V_K) == 0).astype(
    np.int32)[None, :]

_INV_PI = float(1.0 / np.pi)
_PI = float(np.pi)


def _fast_sin(a):
    q = jnp.round(a * _INV_PI)
    r = a - q * _PI
    r2 = r * r
    p = r * (1.0 + r2 * (-0.166666667 + r2 * (0.0083333310
                                              + r2 * (-0.000198408))))
    sign = jax.lax.shift_left(q.astype(jnp.int32) & 1, 31)
    return jax.lax.bitcast_convert_type(
        jax.lax.bitcast_convert_type(p, jnp.int32) ^ sign, jnp.float32)


def _row(ref):
    return jnp.concatenate([ref[i:i + 1, :] for i in range(T2V_IN)], axis=1)


def _tc_body(x_ref, y_ref, e7_ref, e8_ref, m_ref, w_ref, b_ref, yb_ref,
             g_ref, bW_ref, out_ref):
    xtb = x_ref[:, pl.program_id(0), :]
    lp = (jax.lax.broadcasted_iota(jnp.int32, (LENGTH, 1), 0)
          .astype(jnp.float32) * (1.0 / LENGTH))
    xce = jax.lax.dot_general(
        xtb, e7_ref[...], (((0,), (0,)), ((), ())),
        precision=jax.lax.Precision.HIGHEST) + lp * e8_ref[...]
    a = xce * _row(w_ref) + _row(b_ref)
    s = jnp.where(m_ref[...] != 0, a, _fast_sin(a))
    w1 = bW_ref[1:1 + D_MODEL, :]
    c = jnp.reshape(yb_ref[...], (1, D_MODEL)) + g_ref[1:2, :]
    t = jax.lax.dot(s, w1) + c
    w0 = bW_ref[0:1, :]
    yb = jnp.transpose(y_ref[0])
    for v in range(D_Y):
        yv = yb[:, v:v + 1]
        out_ref[0, v * LENGTH:(v + 1) * LENGTH, :] = t + yv * w0


def _sc_body(tab_ref, vemb_ref, vidx_ref, rep_ref, idx_ref, sem):
    core = jax.lax.axis_index("c")
    sub = jax.lax.axis_index("s")
    sid = core * 16 + sub
    v = jax.lax.rem(sid, D_Y)
    b0 = jax.lax.div(sid, D_Y)
    pltpu.async_copy(tab_ref.at[v], rep_ref.at[0], sem).wait()
    vvec = jnp.broadcast_to(v, (16,)).astype(jnp.int32)

    @pl.loop(0, D_MODEL, step=16)
    def _(i):
        val = rep_ref[0, pl.ds(i, 16)]
        for r in range(1, REP):
            rep_ref[r, pl.ds(i, 16)] = val

    @pl.loop(0, LENGTH, step=16)
    def _(i):
        idx_ref[pl.ds(i, 16)] = vvec

    copies = []
    for j in range(2):
        b = b0 + j * 4
        for k in range(LENGTH // REP):
            copies.append(pltpu.async_copy(
                rep_ref,
                vemb_ref.at[b, pl.ds(v * LENGTH + k * REP, REP), :], sem))
        copies.append(pltpu.async_copy(
            idx_ref, vidx_ref.at[b, pl.ds(v * LENGTH, LENGTH)], sem))
    for cp in copies:
        cp.wait()


def _sc_var_outputs(var_emb_table):
    mesh = plsc.VectorSubcoreMesh(core_axis_name="c", subcore_axis_name="s")
    fn = pl.kernel(
        _sc_body,
        out_type=[
            jax.ShapeDtypeStruct((BS, N_TOK, D_MODEL), jnp.float32),
            jax.ShapeDtypeStruct((BS, N_TOK), jnp.int32),
        ],
        mesh=mesh,
        scratch_types=[
            pltpu.VMEM((REP, D_MODEL), jnp.float32),
            pltpu.VMEM((LENGTH,), jnp.int32),
            pltpu.SemaphoreType.DMA,
        ],
    )
    return fn(var_emb_table)


def kernel(y, x, t2v_weight, t2v_bias, y_emb_W, y_emb_b, var_emb_table,
           given_emb_table):
    var_emb, var_idx = _sc_var_outputs(var_emb_table)
    val_time = pl.pallas_call(
        _tc_body,
        grid=(BS,),
        in_specs=[
            pl.BlockSpec((D_X, BS, LENGTH), lambda b: (0, 0, 0)),
            pl.BlockSpec((1, D_Y, LENGTH), lambda b: (b, 0, 0)),
            pl.BlockSpec((D_X, D_MODEL), lambda b: (0, 0)),
            pl.BlockSpec((1, D_MODEL), lambda b: (0, 0)),
            pl.BlockSpec((1, D_MODEL), lambda b: (0, 0)),
            pl.BlockSpec((T2V_IN, T2V_K), lambda b: (0, 0)),
            pl.BlockSpec((T2V_IN, T2V_K), lambda b: (0, 0)),
            pl.BlockSpec((D_MODEL,), lambda b: (0,)),
            pl.BlockSpec((2, D_MODEL), lambda b: (0, 0)),
            pl.BlockSpec((1 + D_MODEL, D_MODEL), lambda b: (0, 0)),
        ],
        out_specs=pl.BlockSpec((1, N_TOK, D_MODEL), lambda b: (b, 0, 0)),
        out_shape=jax.ShapeDtypeStruct((BS, N_TOK, D_MODEL), jnp.float32),
    )(jnp.transpose(x, (2, 0, 1)), jnp.transpose(y, (0, 2, 1)),
      _E7, _E8, _LIN_MASK, t2v_weight, t2v_bias, y_emb_b, given_emb_table,
      y_emb_W)
    return val_time, var_emb, var_idx
